# baseline (speedup 1.0000x reference)
import functools

import jax
import jax.numpy as jnp
from jax import lax
from jax.experimental import pallas as pl
from jax.experimental.pallas import tpu as pltpu
from jax.experimental.pallas import tpu_sc as plsc

_B = 16
_N = 20000
_L = 12
_HALF = _N // 2
_STEPS = _HALF // 16

_UNROLL = 5
_NUM_NEG_RATIO = 3
_UPPER_CONF = 0.8
_LOWER_CONF = 0.3
_UPPER_IOU = 0.5
_LOWER_IOU = 0.35
_REG_COEFF = 1.0
_MAX_NUM_OBJS = 10

_REC = 144


def _splat(v, dtype=jnp.int32):
    return jnp.full((16,), v, dtype)


def _sc_body(outs_hbm, labels_hbm, outf_hbm, cx_s, cy_s, w_s, h_s, lg_s,
             cxc, cyc, wc, hc, lgc, flgc, flgt, slotlog, outv, labv, dsem,
             dsem2):
    img = lax.axis_index("s")
    half = lax.axis_index("c")
    sl = pl.ds(half * _HALF, _HALF)
    cp_lg = pltpu.async_copy(outs_hbm.at[4, img, sl], lg_s, dsem2)
    cps = [
        pltpu.async_copy(outs_hbm.at[0, img, sl], cx_s, dsem),
        pltpu.async_copy(outs_hbm.at[1, img, sl], cy_s, dsem),
        pltpu.async_copy(outs_hbm.at[2, img, sl], w_s, dsem),
        pltpu.async_copy(outs_hbm.at[3, img, sl], h_s, dsem),
    ]
    pltpu.sync_copy(labels_hbm.at[img], labv)
    cp_lg.wait()

    iot = lax.iota(jnp.int32, 16)
    zeros16 = jnp.zeros((16,), jnp.float32)
    for r in range(9):
        outv[pl.ds(r * 16, 16)] = zeros16

    fill = lg_s[pl.ds(_HALF - 16, 16)][15]
    slotlog[...] = jnp.full((16,), fill)

    def p0a(t, _):
        b0 = t * (16 * _UNROLL)
        for u in range(_UNROLL):
            b = b0 + u * 16
            lg = lg_s[pl.ds(b, 16)]
            conf = 1.0 / (1.0 + jnp.exp(-lg))
            up = conf > _UPPER_CONF
            low = conf < _LOWER_CONF
            flgt[pl.ds(b, 16)] = (
                up.astype(jnp.int32) + 2 * low.astype(jnp.int32)
            )
        return 0

    lax.fori_loop(0, _STEPS // _UNROLL, p0a, 0)
    for cp in cps:
        cp.wait()

    def p0b(t, carry):
        cu, ck = carry
        b0 = t * (16 * _UNROLL)
        for u in range(_UNROLL):
            b = b0 + u * 16
            src = pl.ds(b, 16)
            fl = flgt[src]
            sel = fl != 0
            dst = pl.ds(ck, 16)
            plsc.store_compressed(cxc.at[dst], cx_s[src], mask=sel)
            plsc.store_compressed(cyc.at[dst], cy_s[src], mask=sel)
            plsc.store_compressed(wc.at[dst], w_s[src], mask=sel)
            plsc.store_compressed(hc.at[dst], h_s[src], mask=sel)
            plsc.store_compressed(lgc.at[dst], lg_s[src], mask=sel)
            plsc.store_compressed(flgc.at[dst], fl, mask=sel)
            cu = cu + plsc.all_reduce_population_count((fl & 1) != 0)[0]
            ck = ck + plsc.all_reduce_population_count(sel)[0]
        return cu, ck

    cu, ck = lax.fori_loop(
        0, _STEPS // _UNROLL, p0b, (jnp.int32(0), jnp.int32(0))
    )
    zpad = pl.ds(ck, 16)
    cxc[zpad] = zeros16
    cyc[zpad] = zeros16
    wc[zpad] = zeros16
    hc[zpad] = zeros16
    lgc[zpad] = zeros16
    flgc[zpad] = jnp.zeros((16,), jnp.int32)
    nblk = (ck + 15) >> 4

    def bcond(st):
        t, cnt = st
        return (t < nblk) & (cnt < 12)

    def bbody(st):
        t, cnt = st
        fl = flgc[pl.ds(t * 16, 16)]
        lowm = (fl & 2) != 0
        li = lowm.astype(jnp.int32)
        cs = plsc.cumsum(li)
        rr = (cnt - li) + cs
        scm = lowm & (rr < 12)
        lgv = lgc[pl.ds(t * 16, 16)]
        plsc.store_scatter(slotlog, [jnp.minimum(rr, 15)], lgv, mask=scm)
        return t + 1, cnt + plsc.all_reduce_population_count(lowm)[0]

    _, cl = lax.while_loop(bcond, bbody, (jnp.int32(0), jnp.int32(0)))
    cl = jnp.minimum(cl, jnp.int32(12))

    l12 = jnp.minimum(iot, 11)
    lbx = plsc.load_gather(labv, [l12, _splat(0)])
    lby = plsc.load_gather(labv, [l12, _splat(1)])
    lbw = plsc.load_gather(labv, [l12, _splat(2)])
    lbh = plsc.load_gather(labv, [l12, _splat(3)])
    bxlov = lbx - lbw * 0.5
    bxhiv = lbx + lbw * 0.5
    bylov = lby - lbh * 0.5
    byhiv = lby + lbh * 0.5
    bav = lbw * lbh

    zlane = (lbh == 0.0) & (iot < _L)
    last_idx = jnp.min(jnp.where(zlane, iot, jnp.int32(_L)))
    nlab = jnp.where(last_idx <= _MAX_NUM_OBJS, last_idx, jnp.int32(0))

    def _label_iou(j, b, iot_, xlo, xhi, ylo, yhi, ar, upm, lom, m, g, ml):
        tlx = jnp.maximum(xlo, bxlov[j])
        brx = jnp.minimum(xhi, bxhiv[j])
        tly = jnp.maximum(ylo, bylov[j])
        bry = jnp.minimum(yhi, byhiv[j])
        en = (tlx < brx) & (tly < bry)
        inter = jnp.where(en, (brx - tlx) * (bry - tly), 0.0)
        iou = inter / ((ar + bav[j]) - inter)
        iu = jnp.where(upm, iou, -1.0)
        il = jnp.where(lom, iou, -1.0)
        upd = iu > m
        return (
            jnp.where(upd, iu, m),
            jnp.where(upd, b + iot_, g),
            jnp.maximum(ml, il),
        )

    def _emit_row(j, m, g, ml):
        mstar = jnp.max(m)
        gstar = jnp.min(jnp.where(m == mstar, g, jnp.int32(_HALF)))
        gv = jnp.full((16,), gstar, jnp.int32)
        wlg = jnp.max(plsc.load_gather(lgc, [gv]))
        wcx = jnp.max(plsc.load_gather(cxc, [gv]))
        wcy = jnp.max(plsc.load_gather(cyc, [gv]))
        ww = jnp.max(plsc.load_gather(wc, [gv]))
        wh = jnp.max(plsc.load_gather(hc, [gv]))
        vals = (
            jnp.where(iot == 0, mstar, 0.0)
            + jnp.where(iot == 1, jnp.max(ml), 0.0)
            + jnp.where(iot == 2, wlg, 0.0)
            + jnp.where(iot == 3, wcx, 0.0)
            + jnp.where(iot == 4, wcy, 0.0)
            + jnp.where(iot == 5, ww, 0.0)
            + jnp.where(iot == 6, wh, 0.0)
        )
        plsc.store_scatter(
            outv, [jnp.minimum(iot, 8) * 16 + j], vals, mask=iot < 7
        )

    _init1 = (
        jnp.full((16,), -2.0, jnp.float32),
        jnp.zeros((16,), jnp.int32),
        jnp.full((16,), -1.0, jnp.float32),
    )

    @pl.when(nlab == 6)
    def _():
        def fbody(t, carry):
            b = t * 16
            cx = cxc[pl.ds(b, 16)]
            cy = cyc[pl.ds(b, 16)]
            w = wc[pl.ds(b, 16)]
            h = hc[pl.ds(b, 16)]
            xlo = cx - w * 0.5
            xhi = cx + w * 0.5
            ylo = cy - h * 0.5
            yhi = cy + h * 0.5
            ar = w * h
            fl = flgc[pl.ds(b, 16)]
            upm = (fl & 1) != 0
            lom = (fl & 2) != 0
            out = []
            for j in range(6):
                out.extend(
                    _label_iou(
                        j, b, iot, xlo, xhi, ylo, yhi, ar, upm, lom,
                        *carry[3 * j : 3 * j + 3]
                    )
                )
            return tuple(out)

        acc = lax.fori_loop(0, nblk, fbody, _init1 * 6)
        for j in range(6):
            _emit_row(j, acc[3 * j], acc[3 * j + 1], acc[3 * j + 2])

    @pl.when(nlab != 6)
    def _():
        for j in range(_L):

            @pl.when(jnp.int32(j) < nlab)
            def _(j=j):
                def lbody(t, carry):
                    b = t * 16
                    cx = cxc[pl.ds(b, 16)]
                    cy = cyc[pl.ds(b, 16)]
                    w = wc[pl.ds(b, 16)]
                    h = hc[pl.ds(b, 16)]
                    xlo = cx - w * 0.5
                    xhi = cx + w * 0.5
                    ylo = cy - h * 0.5
                    yhi = cy + h * 0.5
                    ar = w * h
                    fl = flgc[pl.ds(b, 16)]
                    upm = (fl & 1) != 0
                    lom = (fl & 2) != 0
                    return _label_iou(
                        j, b, iot, xlo, xhi, ylo, yhi, ar, upm, lom, *carry
                    )

                m, g, ml = lax.fori_loop(0, nblk, lbody, _init1)
                _emit_row(j, m, g, ml)

    outv[pl.ds(7 * 16, 16)] = slotlog[...]
    outv[pl.ds(8 * 16, 16)] = jnp.where(
        iot == 0, cu.astype(jnp.float32), 0.0
    ) + jnp.where(iot == 1, cl.astype(jnp.float32), 0.0)
    pltpu.sync_copy(outv, outf_hbm.at[half, img])


@functools.cache
def _get_sc_pass():
    return pl.kernel(
        _sc_body,
        out_type=jax.ShapeDtypeStruct((2, _B, _REC), jnp.float32),
        mesh=plsc.VectorSubcoreMesh(
            core_axis_name="c", subcore_axis_name="s", num_cores=2, num_subcores=16
        ),
        compiler_params=pltpu.CompilerParams(
            needs_layout_passes=False, use_tc_tiling_on_sc=False
        ),
        scratch_types=[
            pltpu.VMEM((_HALF,), jnp.float32),
            pltpu.VMEM((_HALF,), jnp.float32),
            pltpu.VMEM((_HALF,), jnp.float32),
            pltpu.VMEM((_HALF,), jnp.float32),
            pltpu.VMEM((_HALF,), jnp.float32),
            pltpu.VMEM((_HALF + 16,), jnp.float32),
            pltpu.VMEM((_HALF + 16,), jnp.float32),
            pltpu.VMEM((_HALF + 16,), jnp.float32),
            pltpu.VMEM((_HALF + 16,), jnp.float32),
            pltpu.VMEM((_HALF + 16,), jnp.float32),
            pltpu.VMEM((_HALF + 16,), jnp.int32),
            pltpu.VMEM((_HALF,), jnp.int32),
            pltpu.VMEM((16,), jnp.float32),
            pltpu.VMEM((_REC,), jnp.float32),
            pltpu.VMEM((_L, 4), jnp.float32),
            pltpu.SemaphoreType.DMA,
            pltpu.SemaphoreType.DMA,
        ],
    )


def _bce(x, t):
    return jnp.maximum(x, 0.0) - x * t + jnp.log1p(jnp.exp(-jnp.abs(x)))


def _sl1(d):
    ad = jnp.abs(d)
    return jnp.where(ad < 1.0, 0.5 * d * d, ad - 0.5)


def _epi_body(lab_ref, f_ref, out_ref):
    l0 = lab_ref[0]
    l1 = lab_ref[1]
    l2 = lab_ref[2]
    l3 = lab_ref[3]
    A = f_ref[0]
    Bh = f_ref[1]
    iotl = lax.broadcasted_iota(jnp.int32, (_B, _L), 1)
    zero = l3 == 0.0
    last_idx = jnp.min(jnp.where(zero, iotl, _L), axis=1)
    gt_valid = iotl < last_idx[:, None]

    muA = A[:, 0:_L]
    muB = Bh[:, 0:_L]
    mlA = A[:, 16 : 16 + _L]
    mlB = Bh[:, 16 : 16 + _L]
    lgA = A[:, 32 : 32 + _L]
    lgB = Bh[:, 32 : 32 + _L]
    slA = A[:, 112 : 112 + _L]
    slB = Bh[:, 112 : 112 + _L]
    cuA = A[:, 128]
    cuB = Bh[:, 128]
    clA = A[:, 129]
    clB = Bh[:, 129]

    condA = muA >= muB
    pos_ious = jnp.where(condA, muA, muB)
    pos_logit = jnp.where(condA, lgA, lgB)
    neg_ious = jnp.maximum(mlA, mlB)
    any_up = (cuA + cuB) > 0.0
    any_low = (clA + clB) > 0.0
    img_ok = (last_idx <= _MAX_NUM_OBJS) & any_up & any_low

    pos_mask = (pos_ious >= _UPPER_IOU) & gt_valid & img_ok[:, None]
    pos_f = pos_mask.astype(jnp.float32)
    num_poses = jnp.sum(pos_f, axis=1)

    cAc = jnp.minimum(clA, 12.0)
    sidx = iotl.astype(jnp.float32)
    shift = sidx - cAc[:, None]
    tio = lax.broadcasted_iota(jnp.int32, (_B, _L, _L), 2).astype(jnp.float32)
    bsel = jnp.sum(
        jnp.where(tio == shift[:, :, None], slB[:, None, :], 0.0), axis=2
    )
    slot_logit = jnp.where(sidx < cAc[:, None], slA, bsel)
    slot_conf = 1.0 / (1.0 + jnp.exp(-slot_logit))

    cand_mask = (
        (neg_ious <= _LOWER_IOU)
        & gt_valid
        & img_ok[:, None]
        & (num_poses > 0.0)[:, None]
    )
    key = jnp.where(cand_mask, -slot_conf, jnp.inf)
    kj = key[:, :, None]
    kk = key[:, None, :]
    jj = lax.broadcasted_iota(jnp.int32, (_B, _L, _L), 1)
    kkx = lax.broadcasted_iota(jnp.int32, (_B, _L, _L), 2)
    rank = jnp.sum(
        ((kk < kj) | ((kk == kj) & (kkx < jj))).astype(jnp.float32), axis=2
    )
    neg_sel = cand_mask & (rank < _NUM_NEG_RATIO * num_poses[:, None])

    conf_loss = jnp.sum(_bce(pos_logit, 1.0) * pos_f) + jnp.sum(
        _bce(slot_logit, 0.0) * neg_sel.astype(jnp.float32)
    )

    max_lens = jnp.maximum(l2, l3)
    safe = jnp.where(pos_mask, max_lens, 1.0)
    reg = (
        _sl1(jnp.where(condA, A[:, 48 : 48 + _L], Bh[:, 48 : 48 + _L]) - l0)
        + _sl1(jnp.where(condA, A[:, 64 : 64 + _L], Bh[:, 64 : 64 + _L]) - l1)
        + _sl1(jnp.where(condA, A[:, 80 : 80 + _L], Bh[:, 80 : 80 + _L]) - l2)
        + _sl1(jnp.where(condA, A[:, 96 : 96 + _L], Bh[:, 96 : 96 + _L]) - l3)
    )
    reg_loss = jnp.sum(reg / safe * pos_f)

    ps = jnp.sum(num_poses)
    has_pos = ps >= 1.0
    denom = jnp.maximum(ps, 1.0)
    loss = (conf_loss + _REG_COEFF * reg_loss) / denom
    z = jnp.float32(0.0)
    o0 = jnp.where(has_pos, loss, z)
    o1 = jnp.where(has_pos, conf_loss / denom, z)
    o2 = jnp.where(has_pos, reg_loss / denom, z)
    o3 = jnp.where(has_pos, ps, z)

    li = lax.broadcasted_iota(jnp.int32, (8, 128), 1)
    out_ref[...] = (
        jnp.where(li == 0, o0, 0.0)
        + jnp.where(li == 1, o1, 0.0)
        + jnp.where(li == 2, o2, 0.0)
        + jnp.where(li == 3, o3, 0.0)
    )


def kernel(labels, outputs):
    outs_t = outputs.transpose(2, 0, 1)
    outf = _get_sc_pass()(outs_t, labels)
    lab_t = labels.transpose(2, 0, 1)
    o = pl.pallas_call(
        _epi_body,
        out_shape=jax.ShapeDtypeStruct((8, 128), jnp.float32),
    )(lab_t, outf)
    return o[0, 0], o[0, 1], o[0, 2], o[0, 3]


# speedup vs baseline: 1.2179x; 1.2179x over previous
import functools

import jax
import jax.numpy as jnp
from jax import lax
from jax.experimental import pallas as pl
from jax.experimental.pallas import tpu as pltpu
from jax.experimental.pallas import tpu_sc as plsc

_B = 16
_N = 20000
_L = 12
_HALF = _N // 2
_STEPS = _HALF // 16

_UNROLL = 5
_NUM_NEG_RATIO = 3
_UPPER_CONF = 0.8
_LOWER_CONF = 0.3
_UPPER_IOU = 0.5
_LOWER_IOU = 0.35
_REG_COEFF = 1.0
_MAX_NUM_OBJS = 10

_REC = 144


def _splat(v, dtype=jnp.int32):
    return jnp.full((16,), v, dtype)


def _sc_body(outs_hbm, labels_hbm, outf_hbm, cx_s, cy_s, w_s, h_s, lg_s,
             cxc, cyc, wc, hc, lgc, flgc, flgt, slotlog, outv, labv, dsem,
             dsem2):
    img = lax.axis_index("s")
    half = lax.axis_index("c")
    sl = pl.ds(half * _HALF, _HALF)
    cp_lg = pltpu.async_copy(outs_hbm.at[4, img, sl], lg_s, dsem2)
    cps = [
        pltpu.async_copy(outs_hbm.at[0, img, sl], cx_s, dsem),
        pltpu.async_copy(outs_hbm.at[1, img, sl], cy_s, dsem),
        pltpu.async_copy(outs_hbm.at[2, img, sl], w_s, dsem),
        pltpu.async_copy(outs_hbm.at[3, img, sl], h_s, dsem),
    ]
    pltpu.sync_copy(labels_hbm.at[img], labv)
    cp_lg.wait()

    iot = lax.iota(jnp.int32, 16)
    zeros16 = jnp.zeros((16,), jnp.float32)
    for r in range(9):
        outv[pl.ds(r * 16, 16)] = zeros16

    fill = lg_s[pl.ds(_HALF - 16, 16)][15]
    slotlog[...] = jnp.full((16,), fill)

    def p0a(t, _):
        b0 = t * (16 * _UNROLL)
        for u in range(_UNROLL):
            b = b0 + u * 16
            lg = lg_s[pl.ds(b, 16)]
            conf = 1.0 / (1.0 + jnp.exp(-lg))
            up = conf > _UPPER_CONF
            low = conf < _LOWER_CONF
            flgt[pl.ds(b, 16)] = (
                up.astype(jnp.int32) + 2 * low.astype(jnp.int32)
            )
        return 0

    lax.fori_loop(0, _STEPS // _UNROLL, p0a, 0)
    for cp in cps:
        cp.wait()

    def p0b(t, carry):
        cu, ck = carry
        b0 = t * (16 * _UNROLL)
        for u in range(_UNROLL):
            b = b0 + u * 16
            src = pl.ds(b, 16)
            fl = flgt[src]
            sel = fl != 0
            dst = pl.ds(ck, 16)
            plsc.store_compressed(cxc.at[dst], cx_s[src], mask=sel)
            plsc.store_compressed(cyc.at[dst], cy_s[src], mask=sel)
            plsc.store_compressed(wc.at[dst], w_s[src], mask=sel)
            plsc.store_compressed(hc.at[dst], h_s[src], mask=sel)
            plsc.store_compressed(lgc.at[dst], lg_s[src], mask=sel)
            plsc.store_compressed(flgc.at[dst], fl, mask=sel)
            cu = cu + plsc.all_reduce_population_count((fl & 1) != 0)[0]
            ck = ck + plsc.all_reduce_population_count(sel)[0]
        return cu, ck

    cu, ck = lax.fori_loop(
        0, _STEPS // _UNROLL, p0b, (jnp.int32(0), jnp.int32(0))
    )
    zpad = pl.ds(ck, 16)
    cxc[zpad] = zeros16
    cyc[zpad] = zeros16
    wc[zpad] = zeros16
    hc[zpad] = zeros16
    lgc[zpad] = zeros16
    flgc[zpad] = jnp.zeros((16,), jnp.int32)
    nblk = (ck + 15) >> 4

    def bcond(st):
        t, cnt = st
        return (t < nblk) & (cnt < 12)

    def bbody(st):
        t, cnt = st
        fl = flgc[pl.ds(t * 16, 16)]
        lowm = (fl & 2) != 0
        li = lowm.astype(jnp.int32)
        cs = plsc.cumsum(li)
        rr = (cnt - li) + cs
        scm = lowm & (rr < 12)
        lgv = lgc[pl.ds(t * 16, 16)]
        plsc.store_scatter(slotlog, [jnp.minimum(rr, 15)], lgv, mask=scm)
        return t + 1, cnt + plsc.all_reduce_population_count(lowm)[0]

    _, cl = lax.while_loop(bcond, bbody, (jnp.int32(0), jnp.int32(0)))
    cl = jnp.minimum(cl, jnp.int32(12))

    l12 = jnp.minimum(iot, 11)
    lbx = plsc.load_gather(labv, [l12, _splat(0)])
    lby = plsc.load_gather(labv, [l12, _splat(1)])
    lbw = plsc.load_gather(labv, [l12, _splat(2)])
    lbh = plsc.load_gather(labv, [l12, _splat(3)])
    bxlov = lbx - lbw * 0.5
    bxhiv = lbx + lbw * 0.5
    bylov = lby - lbh * 0.5
    byhiv = lby + lbh * 0.5
    bav = lbw * lbh

    zlane = (lbh == 0.0) & (iot < _L)
    last_idx = jnp.min(jnp.where(zlane, iot, jnp.int32(_L)))
    nlab = jnp.where(last_idx <= _MAX_NUM_OBJS, last_idx, jnp.int32(0))

    def _label_iou(j, b, iot_, xlo, xhi, ylo, yhi, ar, upm, lom, m, g, ml):
        tlx = jnp.maximum(xlo, bxlov[j])
        brx = jnp.minimum(xhi, bxhiv[j])
        tly = jnp.maximum(ylo, bylov[j])
        bry = jnp.minimum(yhi, byhiv[j])
        en = (tlx < brx) & (tly < bry)
        inter = jnp.where(en, (brx - tlx) * (bry - tly), 0.0)
        iou = inter / ((ar + bav[j]) - inter)
        iu = jnp.where(upm, iou, -1.0)
        il = jnp.where(lom, iou, -1.0)
        upd = iu > m
        return (
            jnp.where(upd, iu, m),
            jnp.where(upd, b + iot_, g),
            jnp.maximum(ml, il),
        )

    def _emit_row(j, m, g, ml):
        mstar = jnp.max(m)
        gstar = jnp.min(jnp.where(m == mstar, g, jnp.int32(_HALF)))
        gv = jnp.full((16,), gstar, jnp.int32)
        wlg = jnp.max(plsc.load_gather(lgc, [gv]))
        wcx = jnp.max(plsc.load_gather(cxc, [gv]))
        wcy = jnp.max(plsc.load_gather(cyc, [gv]))
        ww = jnp.max(plsc.load_gather(wc, [gv]))
        wh = jnp.max(plsc.load_gather(hc, [gv]))
        vals = (
            jnp.where(iot == 0, mstar, 0.0)
            + jnp.where(iot == 1, jnp.max(ml), 0.0)
            + jnp.where(iot == 2, wlg, 0.0)
            + jnp.where(iot == 3, wcx, 0.0)
            + jnp.where(iot == 4, wcy, 0.0)
            + jnp.where(iot == 5, ww, 0.0)
            + jnp.where(iot == 6, wh, 0.0)
        )
        plsc.store_scatter(
            outv, [jnp.minimum(iot, 8) * 16 + j], vals, mask=iot < 7
        )

    _init1 = (
        jnp.full((16,), -2.0, jnp.float32),
        jnp.zeros((16,), jnp.int32),
        jnp.full((16,), -1.0, jnp.float32),
    )

    for j in range(_L):

        @pl.when(jnp.int32(j) < nlab)
        def _(j=j):
            def lbody(t, carry):
                b = t * 16
                cx = cxc[pl.ds(b, 16)]
                cy = cyc[pl.ds(b, 16)]
                w = wc[pl.ds(b, 16)]
                h = hc[pl.ds(b, 16)]
                xlo = cx - w * 0.5
                xhi = cx + w * 0.5
                ylo = cy - h * 0.5
                yhi = cy + h * 0.5
                ar = w * h
                fl = flgc[pl.ds(b, 16)]
                upm = (fl & 1) != 0
                lom = (fl & 2) != 0
                return _label_iou(
                    j, b, iot, xlo, xhi, ylo, yhi, ar, upm, lom, *carry
                )

            m, g, ml = lax.fori_loop(0, nblk, lbody, _init1)
            _emit_row(j, m, g, ml)

    outv[pl.ds(7 * 16, 16)] = slotlog[...]
    outv[pl.ds(8 * 16, 16)] = jnp.where(
        iot == 0, cu.astype(jnp.float32), 0.0
    ) + jnp.where(iot == 1, cl.astype(jnp.float32), 0.0)
    pltpu.sync_copy(outv, outf_hbm.at[half, img])


@functools.cache
def _get_sc_pass():
    return pl.kernel(
        _sc_body,
        out_type=jax.ShapeDtypeStruct((2, _B, _REC), jnp.float32),
        mesh=plsc.VectorSubcoreMesh(
            core_axis_name="c", subcore_axis_name="s", num_cores=2, num_subcores=16
        ),
        compiler_params=pltpu.CompilerParams(
            needs_layout_passes=False, use_tc_tiling_on_sc=False
        ),
        scratch_types=[
            pltpu.VMEM((_HALF,), jnp.float32),
            pltpu.VMEM((_HALF,), jnp.float32),
            pltpu.VMEM((_HALF,), jnp.float32),
            pltpu.VMEM((_HALF,), jnp.float32),
            pltpu.VMEM((_HALF,), jnp.float32),
            pltpu.VMEM((_HALF + 16,), jnp.float32),
            pltpu.VMEM((_HALF + 16,), jnp.float32),
            pltpu.VMEM((_HALF + 16,), jnp.float32),
            pltpu.VMEM((_HALF + 16,), jnp.float32),
            pltpu.VMEM((_HALF + 16,), jnp.float32),
            pltpu.VMEM((_HALF + 16,), jnp.int32),
            pltpu.VMEM((_HALF,), jnp.int32),
            pltpu.VMEM((16,), jnp.float32),
            pltpu.VMEM((_REC,), jnp.float32),
            pltpu.VMEM((_L, 4), jnp.float32),
            pltpu.SemaphoreType.DMA,
            pltpu.SemaphoreType.DMA,
        ],
    )


def _bce(x, t):
    return jnp.maximum(x, 0.0) - x * t + jnp.log1p(jnp.exp(-jnp.abs(x)))


def _sl1(d):
    ad = jnp.abs(d)
    return jnp.where(ad < 1.0, 0.5 * d * d, ad - 0.5)


def _epi_body(lab_ref, f_ref, out_ref):
    l0 = lab_ref[0]
    l1 = lab_ref[1]
    l2 = lab_ref[2]
    l3 = lab_ref[3]
    A = f_ref[0]
    Bh = f_ref[1]
    iotl = lax.broadcasted_iota(jnp.int32, (_B, _L), 1)
    zero = l3 == 0.0
    last_idx = jnp.min(jnp.where(zero, iotl, _L), axis=1)
    gt_valid = iotl < last_idx[:, None]

    muA = A[:, 0:_L]
    muB = Bh[:, 0:_L]
    mlA = A[:, 16 : 16 + _L]
    mlB = Bh[:, 16 : 16 + _L]
    lgA = A[:, 32 : 32 + _L]
    lgB = Bh[:, 32 : 32 + _L]
    slA = A[:, 112 : 112 + _L]
    slB = Bh[:, 112 : 112 + _L]
    cuA = A[:, 128]
    cuB = Bh[:, 128]
    clA = A[:, 129]
    clB = Bh[:, 129]

    condA = muA >= muB
    pos_ious = jnp.where(condA, muA, muB)
    pos_logit = jnp.where(condA, lgA, lgB)
    neg_ious = jnp.maximum(mlA, mlB)
    any_up = (cuA + cuB) > 0.0
    any_low = (clA + clB) > 0.0
    img_ok = (last_idx <= _MAX_NUM_OBJS) & any_up & any_low

    pos_mask = (pos_ious >= _UPPER_IOU) & gt_valid & img_ok[:, None]
    pos_f = pos_mask.astype(jnp.float32)
    num_poses = jnp.sum(pos_f, axis=1)

    cAc = jnp.minimum(clA, 12.0)
    sidx = iotl.astype(jnp.float32)
    shift = sidx - cAc[:, None]
    tio = lax.broadcasted_iota(jnp.int32, (_B, _L, _L), 2).astype(jnp.float32)
    bsel = jnp.sum(
        jnp.where(tio == shift[:, :, None], slB[:, None, :], 0.0), axis=2
    )
    slot_logit = jnp.where(sidx < cAc[:, None], slA, bsel)
    slot_conf = 1.0 / (1.0 + jnp.exp(-slot_logit))

    cand_mask = (
        (neg_ious <= _LOWER_IOU)
        & gt_valid
        & img_ok[:, None]
        & (num_poses > 0.0)[:, None]
    )
    key = jnp.where(cand_mask, -slot_conf, jnp.inf)
    kj = key[:, :, None]
    kk = key[:, None, :]
    jj = lax.broadcasted_iota(jnp.int32, (_B, _L, _L), 1)
    kkx = lax.broadcasted_iota(jnp.int32, (_B, _L, _L), 2)
    rank = jnp.sum(
        ((kk < kj) | ((kk == kj) & (kkx < jj))).astype(jnp.float32), axis=2
    )
    neg_sel = cand_mask & (rank < _NUM_NEG_RATIO * num_poses[:, None])

    conf_loss = jnp.sum(_bce(pos_logit, 1.0) * pos_f) + jnp.sum(
        _bce(slot_logit, 0.0) * neg_sel.astype(jnp.float32)
    )

    max_lens = jnp.maximum(l2, l3)
    safe = jnp.where(pos_mask, max_lens, 1.0)
    reg = (
        _sl1(jnp.where(condA, A[:, 48 : 48 + _L], Bh[:, 48 : 48 + _L]) - l0)
        + _sl1(jnp.where(condA, A[:, 64 : 64 + _L], Bh[:, 64 : 64 + _L]) - l1)
        + _sl1(jnp.where(condA, A[:, 80 : 80 + _L], Bh[:, 80 : 80 + _L]) - l2)
        + _sl1(jnp.where(condA, A[:, 96 : 96 + _L], Bh[:, 96 : 96 + _L]) - l3)
    )
    reg_loss = jnp.sum(reg / safe * pos_f)

    ps = jnp.sum(num_poses)
    has_pos = ps >= 1.0
    denom = jnp.maximum(ps, 1.0)
    loss = (conf_loss + _REG_COEFF * reg_loss) / denom
    z = jnp.float32(0.0)
    o0 = jnp.where(has_pos, loss, z)
    o1 = jnp.where(has_pos, conf_loss / denom, z)
    o2 = jnp.where(has_pos, reg_loss / denom, z)
    o3 = jnp.where(has_pos, ps, z)

    li = lax.broadcasted_iota(jnp.int32, (8, 128), 1)
    out_ref[...] = (
        jnp.where(li == 0, o0, 0.0)
        + jnp.where(li == 1, o1, 0.0)
        + jnp.where(li == 2, o2, 0.0)
        + jnp.where(li == 3, o3, 0.0)
    )


def kernel(labels, outputs):
    outs_t = outputs.transpose(2, 0, 1)
    outf = _get_sc_pass()(outs_t, labels)
    lab_t = labels.transpose(2, 0, 1)
    o = pl.pallas_call(
        _epi_body,
        out_shape=jax.ShapeDtypeStruct((8, 128), jnp.float32),
    )(lab_t, outf)
    return o[0, 0], o[0, 1], o[0, 2], o[0, 3]


# speedup vs baseline: 1.2719x; 1.0443x over previous
import functools

import jax
import jax.numpy as jnp
from jax import lax
from jax.experimental import pallas as pl
from jax.experimental.pallas import tpu as pltpu
from jax.experimental.pallas import tpu_sc as plsc

_B = 16
_N = 20000
_L = 12
_HALF = _N // 2
_STEPS = _HALF // 16

_UNROLL = 5
_NUM_NEG_RATIO = 3
_UPPER_CONF = 0.8
_LOWER_CONF = 0.3
_UPPER_IOU = 0.5
_LOWER_IOU = 0.35
_REG_COEFF = 1.0
_MAX_NUM_OBJS = 10

_REC = 144


def _splat(v, dtype=jnp.int32):
    return jnp.full((16,), v, dtype)


def _sc_body(outs_hbm, labels_hbm, outf_hbm, cx_s, cy_s, w_s, h_s, lg_s,
             cxc, cyc, wc, hc, lgc, flgc, flgt, slotlog, outv, labv, dsem,
             dsem2):
    img = lax.axis_index("s")
    half = lax.axis_index("c")
    sl = pl.ds(half * _HALF, _HALF)
    cp_lg = pltpu.async_copy(outs_hbm.at[4, img, sl], lg_s, dsem2)
    cps = [
        pltpu.async_copy(outs_hbm.at[0, img, sl], cx_s, dsem),
        pltpu.async_copy(outs_hbm.at[1, img, sl], cy_s, dsem),
        pltpu.async_copy(outs_hbm.at[2, img, sl], w_s, dsem),
        pltpu.async_copy(outs_hbm.at[3, img, sl], h_s, dsem),
    ]
    pltpu.sync_copy(labels_hbm.at[img], labv)
    cp_lg.wait()

    iot = lax.iota(jnp.int32, 16)
    zeros16 = jnp.zeros((16,), jnp.float32)
    for r in range(9):
        outv[pl.ds(r * 16, 16)] = zeros16

    fill = lg_s[pl.ds(_HALF - 16, 16)][15]
    slotlog[...] = jnp.full((16,), fill)

    def p0a(t, _):
        b0 = t * (16 * _UNROLL)
        for u in range(_UNROLL):
            b = b0 + u * 16
            lg = lg_s[pl.ds(b, 16)]
            conf = 1.0 / (1.0 + jnp.exp(-lg))
            up = conf > _UPPER_CONF
            low = conf < _LOWER_CONF
            flgt[pl.ds(b, 16)] = (
                up.astype(jnp.int32) + 2 * low.astype(jnp.int32)
            )
        return 0

    lax.fori_loop(0, _STEPS // _UNROLL, p0a, 0)
    for cp in cps:
        cp.wait()

    def p0b(t, carry):
        cu, ck = carry
        b0 = t * (16 * _UNROLL)
        for u in range(_UNROLL):
            b = b0 + u * 16
            src = pl.ds(b, 16)
            fl = flgt[src]
            sel = fl != 0
            dst = pl.ds(ck, 16)
            plsc.store_compressed(cxc.at[dst], cx_s[src], mask=sel)
            plsc.store_compressed(cyc.at[dst], cy_s[src], mask=sel)
            plsc.store_compressed(wc.at[dst], w_s[src], mask=sel)
            plsc.store_compressed(hc.at[dst], h_s[src], mask=sel)
            plsc.store_compressed(lgc.at[dst], lg_s[src], mask=sel)
            plsc.store_compressed(flgc.at[dst], fl, mask=sel)
            cu = cu + plsc.all_reduce_population_count((fl & 1) != 0)[0]
            ck = ck + plsc.all_reduce_population_count(sel)[0]
        return cu, ck

    cu, ck = lax.fori_loop(
        0, _STEPS // _UNROLL, p0b, (jnp.int32(0), jnp.int32(0))
    )
    zpad = pl.ds(ck, 16)
    cxc[zpad] = zeros16
    cyc[zpad] = zeros16
    wc[zpad] = zeros16
    hc[zpad] = zeros16
    lgc[zpad] = zeros16
    flgc[zpad] = jnp.zeros((16,), jnp.int32)
    nblk = (ck + 15) >> 4

    def bcond(st):
        t, cnt = st
        return (t < nblk) & (cnt < 12)

    def bbody(st):
        t, cnt = st
        fl = flgc[pl.ds(t * 16, 16)]
        lowm = (fl & 2) != 0
        li = lowm.astype(jnp.int32)
        cs = plsc.cumsum(li)
        rr = (cnt - li) + cs
        scm = lowm & (rr < 12)
        lgv = lgc[pl.ds(t * 16, 16)]
        plsc.store_scatter(slotlog, [jnp.minimum(rr, 15)], lgv, mask=scm)
        return t + 1, cnt + plsc.all_reduce_population_count(lowm)[0]

    _, cl = lax.while_loop(bcond, bbody, (jnp.int32(0), jnp.int32(0)))
    cl = jnp.minimum(cl, jnp.int32(12))

    l12 = jnp.minimum(iot, 11)
    lbx = plsc.load_gather(labv, [l12, _splat(0)])
    lby = plsc.load_gather(labv, [l12, _splat(1)])
    lbw = plsc.load_gather(labv, [l12, _splat(2)])
    lbh = plsc.load_gather(labv, [l12, _splat(3)])
    bxlov = lbx - lbw * 0.5
    bxhiv = lbx + lbw * 0.5
    bylov = lby - lbh * 0.5
    byhiv = lby + lbh * 0.5
    bav = lbw * lbh

    zlane = (lbh == 0.0) & (iot < _L)
    last_idx = jnp.min(jnp.where(zlane, iot, jnp.int32(_L)))
    nlab = jnp.where(last_idx <= _MAX_NUM_OBJS, last_idx, jnp.int32(0))

    def _label_iou(j, b, iot_, xlo, xhi, ylo, yhi, ar, upm, lom, m, g, ml):
        tlx = jnp.maximum(xlo, bxlov[j])
        brx = jnp.minimum(xhi, bxhiv[j])
        tly = jnp.maximum(ylo, bylov[j])
        bry = jnp.minimum(yhi, byhiv[j])
        en = (tlx < brx) & (tly < bry)
        inter = jnp.where(en, (brx - tlx) * (bry - tly), 0.0)
        iou = inter / ((ar + bav[j]) - inter)
        iu = jnp.where(upm, iou, -1.0)
        il = jnp.where(lom, iou, -1.0)
        upd = iu > m
        return (
            jnp.where(upd, iu, m),
            jnp.where(upd, b + iot_, g),
            jnp.maximum(ml, il),
        )

    def _emit_row(j, m, g, ml):
        mstar = jnp.max(m)
        gstar = jnp.min(jnp.where(m == mstar, g, jnp.int32(_HALF)))
        gv = jnp.full((16,), gstar, jnp.int32)
        wlg = jnp.max(plsc.load_gather(lgc, [gv]))
        wcx = jnp.max(plsc.load_gather(cxc, [gv]))
        wcy = jnp.max(plsc.load_gather(cyc, [gv]))
        ww = jnp.max(plsc.load_gather(wc, [gv]))
        wh = jnp.max(plsc.load_gather(hc, [gv]))
        vals = (
            jnp.where(iot == 0, mstar, 0.0)
            + jnp.where(iot == 1, jnp.max(ml), 0.0)
            + jnp.where(iot == 2, wlg, 0.0)
            + jnp.where(iot == 3, wcx, 0.0)
            + jnp.where(iot == 4, wcy, 0.0)
            + jnp.where(iot == 5, ww, 0.0)
            + jnp.where(iot == 6, wh, 0.0)
        )
        plsc.store_scatter(
            outv, [jnp.minimum(iot, 8) * 16 + j], vals, mask=iot < 7
        )

    _init1 = (
        jnp.full((16,), -2.0, jnp.float32),
        jnp.zeros((16,), jnp.int32),
        jnp.full((16,), -1.0, jnp.float32),
    )

    for p in range(_L // 2):
        j0, j1 = 2 * p, 2 * p + 1

        @pl.when(jnp.int32(j0) < nlab)
        def _(j0=j0, j1=j1):
            v1 = jnp.int32(j1) < nlab

            def lbody(t, carry):
                b = t * 16
                cx = cxc[pl.ds(b, 16)]
                cy = cyc[pl.ds(b, 16)]
                w = wc[pl.ds(b, 16)]
                h = hc[pl.ds(b, 16)]
                xlo = cx - w * 0.5
                xhi = cx + w * 0.5
                ylo = cy - h * 0.5
                yhi = cy + h * 0.5
                ar = w * h
                fl = flgc[pl.ds(b, 16)]
                upm = (fl & 1) != 0
                lom = (fl & 2) != 0
                a = _label_iou(
                    j0, b, iot, xlo, xhi, ylo, yhi, ar, upm, lom, *carry[0:3]
                )
                bb = _label_iou(
                    j1, b, iot, xlo, xhi, ylo, yhi, ar, upm & v1, lom & v1,
                    *carry[3:6]
                )
                return a + bb

            acc = lax.fori_loop(0, nblk, lbody, _init1 * 2)
            _emit_row(j0, acc[0], acc[1], acc[2])

            @pl.when(v1)
            def _():
                _emit_row(j1, acc[3], acc[4], acc[5])

    outv[pl.ds(7 * 16, 16)] = slotlog[...]
    outv[pl.ds(8 * 16, 16)] = jnp.where(
        iot == 0, cu.astype(jnp.float32), 0.0
    ) + jnp.where(iot == 1, cl.astype(jnp.float32), 0.0)
    pltpu.sync_copy(outv, outf_hbm.at[half, img])


@functools.cache
def _get_sc_pass():
    return pl.kernel(
        _sc_body,
        out_type=jax.ShapeDtypeStruct((2, _B, _REC), jnp.float32),
        mesh=plsc.VectorSubcoreMesh(
            core_axis_name="c", subcore_axis_name="s", num_cores=2, num_subcores=16
        ),
        compiler_params=pltpu.CompilerParams(
            needs_layout_passes=False, use_tc_tiling_on_sc=False
        ),
        scratch_types=[
            pltpu.VMEM((_HALF,), jnp.float32),
            pltpu.VMEM((_HALF,), jnp.float32),
            pltpu.VMEM((_HALF,), jnp.float32),
            pltpu.VMEM((_HALF,), jnp.float32),
            pltpu.VMEM((_HALF,), jnp.float32),
            pltpu.VMEM((_HALF + 16,), jnp.float32),
            pltpu.VMEM((_HALF + 16,), jnp.float32),
            pltpu.VMEM((_HALF + 16,), jnp.float32),
            pltpu.VMEM((_HALF + 16,), jnp.float32),
            pltpu.VMEM((_HALF + 16,), jnp.float32),
            pltpu.VMEM((_HALF + 16,), jnp.int32),
            pltpu.VMEM((_HALF,), jnp.int32),
            pltpu.VMEM((16,), jnp.float32),
            pltpu.VMEM((_REC,), jnp.float32),
            pltpu.VMEM((_L, 4), jnp.float32),
            pltpu.SemaphoreType.DMA,
            pltpu.SemaphoreType.DMA,
        ],
    )


def _bce(x, t):
    return jnp.maximum(x, 0.0) - x * t + jnp.log1p(jnp.exp(-jnp.abs(x)))


def _sl1(d):
    ad = jnp.abs(d)
    return jnp.where(ad < 1.0, 0.5 * d * d, ad - 0.5)


def _epi_body(lab_ref, f_ref, out_ref):
    l0 = lab_ref[0]
    l1 = lab_ref[1]
    l2 = lab_ref[2]
    l3 = lab_ref[3]
    A = f_ref[0]
    Bh = f_ref[1]
    iotl = lax.broadcasted_iota(jnp.int32, (_B, _L), 1)
    zero = l3 == 0.0
    last_idx = jnp.min(jnp.where(zero, iotl, _L), axis=1)
    gt_valid = iotl < last_idx[:, None]

    muA = A[:, 0:_L]
    muB = Bh[:, 0:_L]
    mlA = A[:, 16 : 16 + _L]
    mlB = Bh[:, 16 : 16 + _L]
    lgA = A[:, 32 : 32 + _L]
    lgB = Bh[:, 32 : 32 + _L]
    slA = A[:, 112 : 112 + _L]
    slB = Bh[:, 112 : 112 + _L]
    cuA = A[:, 128]
    cuB = Bh[:, 128]
    clA = A[:, 129]
    clB = Bh[:, 129]

    condA = muA >= muB
    pos_ious = jnp.where(condA, muA, muB)
    pos_logit = jnp.where(condA, lgA, lgB)
    neg_ious = jnp.maximum(mlA, mlB)
    any_up = (cuA + cuB) > 0.0
    any_low = (clA + clB) > 0.0
    img_ok = (last_idx <= _MAX_NUM_OBJS) & any_up & any_low

    pos_mask = (pos_ious >= _UPPER_IOU) & gt_valid & img_ok[:, None]
    pos_f = pos_mask.astype(jnp.float32)
    num_poses = jnp.sum(pos_f, axis=1)

    cAc = jnp.minimum(clA, 12.0)
    sidx = iotl.astype(jnp.float32)
    shift = sidx - cAc[:, None]
    tio = lax.broadcasted_iota(jnp.int32, (_B, _L, _L), 2).astype(jnp.float32)
    bsel = jnp.sum(
        jnp.where(tio == shift[:, :, None], slB[:, None, :], 0.0), axis=2
    )
    slot_logit = jnp.where(sidx < cAc[:, None], slA, bsel)
    slot_conf = 1.0 / (1.0 + jnp.exp(-slot_logit))

    cand_mask = (
        (neg_ious <= _LOWER_IOU)
        & gt_valid
        & img_ok[:, None]
        & (num_poses > 0.0)[:, None]
    )
    key = jnp.where(cand_mask, -slot_conf, jnp.inf)
    kj = key[:, :, None]
    kk = key[:, None, :]
    jj = lax.broadcasted_iota(jnp.int32, (_B, _L, _L), 1)
    kkx = lax.broadcasted_iota(jnp.int32, (_B, _L, _L), 2)
    rank = jnp.sum(
        ((kk < kj) | ((kk == kj) & (kkx < jj))).astype(jnp.float32), axis=2
    )
    neg_sel = cand_mask & (rank < _NUM_NEG_RATIO * num_poses[:, None])

    conf_loss = jnp.sum(_bce(pos_logit, 1.0) * pos_f) + jnp.sum(
        _bce(slot_logit, 0.0) * neg_sel.astype(jnp.float32)
    )

    max_lens = jnp.maximum(l2, l3)
    safe = jnp.where(pos_mask, max_lens, 1.0)
    reg = (
        _sl1(jnp.where(condA, A[:, 48 : 48 + _L], Bh[:, 48 : 48 + _L]) - l0)
        + _sl1(jnp.where(condA, A[:, 64 : 64 + _L], Bh[:, 64 : 64 + _L]) - l1)
        + _sl1(jnp.where(condA, A[:, 80 : 80 + _L], Bh[:, 80 : 80 + _L]) - l2)
        + _sl1(jnp.where(condA, A[:, 96 : 96 + _L], Bh[:, 96 : 96 + _L]) - l3)
    )
    reg_loss = jnp.sum(reg / safe * pos_f)

    ps = jnp.sum(num_poses)
    has_pos = ps >= 1.0
    denom = jnp.maximum(ps, 1.0)
    loss = (conf_loss + _REG_COEFF * reg_loss) / denom
    z = jnp.float32(0.0)
    o0 = jnp.where(has_pos, loss, z)
    o1 = jnp.where(has_pos, conf_loss / denom, z)
    o2 = jnp.where(has_pos, reg_loss / denom, z)
    o3 = jnp.where(has_pos, ps, z)

    li = lax.broadcasted_iota(jnp.int32, (8, 128), 1)
    out_ref[...] = (
        jnp.where(li == 0, o0, 0.0)
        + jnp.where(li == 1, o1, 0.0)
        + jnp.where(li == 2, o2, 0.0)
        + jnp.where(li == 3, o3, 0.0)
    )


def kernel(labels, outputs):
    outs_t = outputs.transpose(2, 0, 1)
    outf = _get_sc_pass()(outs_t, labels)
    lab_t = labels.transpose(2, 0, 1)
    o = pl.pallas_call(
        _epi_body,
        out_shape=jax.ShapeDtypeStruct((8, 128), jnp.float32),
    )(lab_t, outf)
    return o[0, 0], o[0, 1], o[0, 2], o[0, 3]


# speedup vs baseline: 1.2815x; 1.0076x over previous
import functools

import jax
import jax.numpy as jnp
from jax import lax
from jax.experimental import pallas as pl
from jax.experimental.pallas import tpu as pltpu
from jax.experimental.pallas import tpu_sc as plsc

_B = 16
_N = 20000
_L = 12
_HALF = _N // 2
_STEPS = _HALF // 16

_UNROLL = 5
_NUM_NEG_RATIO = 3
_UPPER_CONF = 0.8
_LOWER_CONF = 0.3
_UPPER_IOU = 0.5
_LOWER_IOU = 0.35
_REG_COEFF = 1.0
_MAX_NUM_OBJS = 10

_REC = 144


def _splat(v, dtype=jnp.int32):
    return jnp.full((16,), v, dtype)


def _sc_body(outs_hbm, labels_hbm, outf_hbm, cx_s, cy_s, w_s, h_s, lg_s,
             cxc, cyc, wc, hc, lgc, flgc, flgt, slotlog, outv, labv, dsem,
             dsem2):
    img = lax.axis_index("s")
    half = lax.axis_index("c")
    sl = pl.ds(half * _HALF, _HALF)
    cp_lg = pltpu.async_copy(outs_hbm.at[4, img, sl], lg_s, dsem2)
    cps = [
        pltpu.async_copy(outs_hbm.at[0, img, sl], cx_s, dsem),
        pltpu.async_copy(outs_hbm.at[1, img, sl], cy_s, dsem),
        pltpu.async_copy(outs_hbm.at[2, img, sl], w_s, dsem),
        pltpu.async_copy(outs_hbm.at[3, img, sl], h_s, dsem),
    ]
    pltpu.sync_copy(labels_hbm.at[img], labv)
    cp_lg.wait()

    iot = lax.iota(jnp.int32, 16)
    zeros16 = jnp.zeros((16,), jnp.float32)
    for r in range(9):
        outv[pl.ds(r * 16, 16)] = zeros16

    fill = lg_s[pl.ds(_HALF - 16, 16)][15]
    slotlog[...] = jnp.full((16,), fill)

    def p0a(t, _):
        b0 = t * (16 * _UNROLL)
        for u in range(_UNROLL):
            b = b0 + u * 16
            lg = lg_s[pl.ds(b, 16)]
            conf = 1.0 / (1.0 + jnp.exp(-lg))
            up = conf > _UPPER_CONF
            low = conf < _LOWER_CONF
            flgt[pl.ds(b, 16)] = (
                up.astype(jnp.int32) + 2 * low.astype(jnp.int32)
            )
        return 0

    lax.fori_loop(0, _STEPS // _UNROLL, p0a, 0)
    for cp in cps:
        cp.wait()

    def p0b(t, carry):
        cu, ck = carry
        b0 = t * (16 * _UNROLL)
        for u in range(_UNROLL):
            b = b0 + u * 16
            src = pl.ds(b, 16)
            fl = flgt[src]
            sel = fl != 0
            dst = pl.ds(ck, 16)
            plsc.store_compressed(cxc.at[dst], cx_s[src], mask=sel)
            plsc.store_compressed(cyc.at[dst], cy_s[src], mask=sel)
            plsc.store_compressed(wc.at[dst], w_s[src], mask=sel)
            plsc.store_compressed(hc.at[dst], h_s[src], mask=sel)
            plsc.store_compressed(lgc.at[dst], lg_s[src], mask=sel)
            plsc.store_compressed(flgc.at[dst], fl, mask=sel)
            cu = cu + plsc.all_reduce_population_count((fl & 1) != 0)[0]
            ck = ck + plsc.all_reduce_population_count(sel)[0]
        return cu, ck

    cu, ck = lax.fori_loop(
        0, _STEPS // _UNROLL, p0b, (jnp.int32(0), jnp.int32(0))
    )
    zpad = pl.ds(ck, 16)
    cxc[zpad] = zeros16
    cyc[zpad] = zeros16
    wc[zpad] = zeros16
    hc[zpad] = zeros16
    lgc[zpad] = zeros16
    flgc[zpad] = jnp.zeros((16,), jnp.int32)
    nblk = (ck + 15) >> 4

    def bcond(st):
        t, cnt = st
        return (t < nblk) & (cnt < 12)

    def bbody(st):
        t, cnt = st
        fl = flgc[pl.ds(t * 16, 16)]
        lowm = (fl & 2) != 0
        li = lowm.astype(jnp.int32)
        cs = plsc.cumsum(li)
        rr = (cnt - li) + cs
        scm = lowm & (rr < 12)
        lgv = lgc[pl.ds(t * 16, 16)]
        plsc.store_scatter(slotlog, [jnp.minimum(rr, 15)], lgv, mask=scm)
        return t + 1, cnt + plsc.all_reduce_population_count(lowm)[0]

    _, cl = lax.while_loop(bcond, bbody, (jnp.int32(0), jnp.int32(0)))
    cl = jnp.minimum(cl, jnp.int32(12))

    l12 = jnp.minimum(iot, 11)
    lbx = plsc.load_gather(labv, [l12, _splat(0)])
    lby = plsc.load_gather(labv, [l12, _splat(1)])
    lbw = plsc.load_gather(labv, [l12, _splat(2)])
    lbh = plsc.load_gather(labv, [l12, _splat(3)])
    bxlov = lbx - lbw * 0.5
    bxhiv = lbx + lbw * 0.5
    bylov = lby - lbh * 0.5
    byhiv = lby + lbh * 0.5
    bav = lbw * lbh

    zlane = (lbh == 0.0) & (iot < _L)
    last_idx = jnp.min(jnp.where(zlane, iot, jnp.int32(_L)))
    nlab = jnp.where(last_idx <= _MAX_NUM_OBJS, last_idx, jnp.int32(0))

    def _label_iou(j, b, iot_, xlo, xhi, ylo, yhi, ar, upm, lom, m, g, ml):
        tlx = jnp.maximum(xlo, bxlov[j])
        brx = jnp.minimum(xhi, bxhiv[j])
        tly = jnp.maximum(ylo, bylov[j])
        bry = jnp.minimum(yhi, byhiv[j])
        en = (tlx < brx) & (tly < bry)
        inter = jnp.where(en, (brx - tlx) * (bry - tly), 0.0)
        iou = inter / ((ar + bav[j]) - inter)
        iu = jnp.where(upm, iou, -1.0)
        il = jnp.where(lom, iou, -1.0)
        upd = iu > m
        return (
            jnp.where(upd, iu, m),
            jnp.where(upd, b + iot_, g),
            jnp.maximum(ml, il),
        )

    def _emit_row(j, m, g, ml):
        mstar = jnp.max(m)
        gstar = jnp.min(jnp.where(m == mstar, g, jnp.int32(_HALF)))
        gv = jnp.full((16,), gstar, jnp.int32)
        wlg = jnp.max(plsc.load_gather(lgc, [gv]))
        wcx = jnp.max(plsc.load_gather(cxc, [gv]))
        wcy = jnp.max(plsc.load_gather(cyc, [gv]))
        ww = jnp.max(plsc.load_gather(wc, [gv]))
        wh = jnp.max(plsc.load_gather(hc, [gv]))
        vals = (
            jnp.where(iot == 0, mstar, 0.0)
            + jnp.where(iot == 1, jnp.max(ml), 0.0)
            + jnp.where(iot == 2, wlg, 0.0)
            + jnp.where(iot == 3, wcx, 0.0)
            + jnp.where(iot == 4, wcy, 0.0)
            + jnp.where(iot == 5, ww, 0.0)
            + jnp.where(iot == 6, wh, 0.0)
        )
        plsc.store_scatter(
            outv, [jnp.minimum(iot, 8) * 16 + j], vals, mask=iot < 7
        )

    _init1 = (
        jnp.full((16,), -2.0, jnp.float32),
        jnp.zeros((16,), jnp.int32),
        jnp.full((16,), -1.0, jnp.float32),
    )

    for p in range(_L // 3):
        j0 = 3 * p

        @pl.when(jnp.int32(j0) < nlab)
        def _(j0=j0):
            v1 = jnp.int32(j0 + 1) < nlab
            v2 = jnp.int32(j0 + 2) < nlab

            def lbody(t, carry):
                b = t * 16
                cx = cxc[pl.ds(b, 16)]
                cy = cyc[pl.ds(b, 16)]
                w = wc[pl.ds(b, 16)]
                h = hc[pl.ds(b, 16)]
                xlo = cx - w * 0.5
                xhi = cx + w * 0.5
                ylo = cy - h * 0.5
                yhi = cy + h * 0.5
                ar = w * h
                fl = flgc[pl.ds(b, 16)]
                upm = (fl & 1) != 0
                lom = (fl & 2) != 0
                a = _label_iou(
                    j0, b, iot, xlo, xhi, ylo, yhi, ar, upm, lom, *carry[0:3]
                )
                bb = _label_iou(
                    j0 + 1, b, iot, xlo, xhi, ylo, yhi, ar, upm & v1,
                    lom & v1, *carry[3:6]
                )
                cc = _label_iou(
                    j0 + 2, b, iot, xlo, xhi, ylo, yhi, ar, upm & v2,
                    lom & v2, *carry[6:9]
                )
                return a + bb + cc

            acc = lax.fori_loop(0, nblk, lbody, _init1 * 3)
            _emit_row(j0, acc[0], acc[1], acc[2])

            @pl.when(v1)
            def _():
                _emit_row(j0 + 1, acc[3], acc[4], acc[5])

            @pl.when(v2)
            def _():
                _emit_row(j0 + 2, acc[6], acc[7], acc[8])

    outv[pl.ds(7 * 16, 16)] = slotlog[...]
    outv[pl.ds(8 * 16, 16)] = jnp.where(
        iot == 0, cu.astype(jnp.float32), 0.0
    ) + jnp.where(iot == 1, cl.astype(jnp.float32), 0.0)
    pltpu.sync_copy(outv, outf_hbm.at[half, img])


@functools.cache
def _get_sc_pass():
    return pl.kernel(
        _sc_body,
        out_type=jax.ShapeDtypeStruct((2, _B, _REC), jnp.float32),
        mesh=plsc.VectorSubcoreMesh(
            core_axis_name="c", subcore_axis_name="s", num_cores=2, num_subcores=16
        ),
        compiler_params=pltpu.CompilerParams(
            needs_layout_passes=False, use_tc_tiling_on_sc=False
        ),
        scratch_types=[
            pltpu.VMEM((_HALF,), jnp.float32),
            pltpu.VMEM((_HALF,), jnp.float32),
            pltpu.VMEM((_HALF,), jnp.float32),
            pltpu.VMEM((_HALF,), jnp.float32),
            pltpu.VMEM((_HALF,), jnp.float32),
            pltpu.VMEM((_HALF + 16,), jnp.float32),
            pltpu.VMEM((_HALF + 16,), jnp.float32),
            pltpu.VMEM((_HALF + 16,), jnp.float32),
            pltpu.VMEM((_HALF + 16,), jnp.float32),
            pltpu.VMEM((_HALF + 16,), jnp.float32),
            pltpu.VMEM((_HALF + 16,), jnp.int32),
            pltpu.VMEM((_HALF,), jnp.int32),
            pltpu.VMEM((16,), jnp.float32),
            pltpu.VMEM((_REC,), jnp.float32),
            pltpu.VMEM((_L, 4), jnp.float32),
            pltpu.SemaphoreType.DMA,
            pltpu.SemaphoreType.DMA,
        ],
    )


def _bce(x, t):
    return jnp.maximum(x, 0.0) - x * t + jnp.log1p(jnp.exp(-jnp.abs(x)))


def _sl1(d):
    ad = jnp.abs(d)
    return jnp.where(ad < 1.0, 0.5 * d * d, ad - 0.5)


def _epi_body(lab_ref, f_ref, out_ref):
    l0 = lab_ref[0]
    l1 = lab_ref[1]
    l2 = lab_ref[2]
    l3 = lab_ref[3]
    A = f_ref[0]
    Bh = f_ref[1]
    iotl = lax.broadcasted_iota(jnp.int32, (_B, _L), 1)
    zero = l3 == 0.0
    last_idx = jnp.min(jnp.where(zero, iotl, _L), axis=1)
    gt_valid = iotl < last_idx[:, None]

    muA = A[:, 0:_L]
    muB = Bh[:, 0:_L]
    mlA = A[:, 16 : 16 + _L]
    mlB = Bh[:, 16 : 16 + _L]
    lgA = A[:, 32 : 32 + _L]
    lgB = Bh[:, 32 : 32 + _L]
    slA = A[:, 112 : 112 + _L]
    slB = Bh[:, 112 : 112 + _L]
    cuA = A[:, 128]
    cuB = Bh[:, 128]
    clA = A[:, 129]
    clB = Bh[:, 129]

    condA = muA >= muB
    pos_ious = jnp.where(condA, muA, muB)
    pos_logit = jnp.where(condA, lgA, lgB)
    neg_ious = jnp.maximum(mlA, mlB)
    any_up = (cuA + cuB) > 0.0
    any_low = (clA + clB) > 0.0
    img_ok = (last_idx <= _MAX_NUM_OBJS) & any_up & any_low

    pos_mask = (pos_ious >= _UPPER_IOU) & gt_valid & img_ok[:, None]
    pos_f = pos_mask.astype(jnp.float32)
    num_poses = jnp.sum(pos_f, axis=1)

    cAc = jnp.minimum(clA, 12.0)
    sidx = iotl.astype(jnp.float32)
    shift = sidx - cAc[:, None]
    tio = lax.broadcasted_iota(jnp.int32, (_B, _L, _L), 2).astype(jnp.float32)
    bsel = jnp.sum(
        jnp.where(tio == shift[:, :, None], slB[:, None, :], 0.0), axis=2
    )
    slot_logit = jnp.where(sidx < cAc[:, None], slA, bsel)
    slot_conf = 1.0 / (1.0 + jnp.exp(-slot_logit))

    cand_mask = (
        (neg_ious <= _LOWER_IOU)
        & gt_valid
        & img_ok[:, None]
        & (num_poses > 0.0)[:, None]
    )
    key = jnp.where(cand_mask, -slot_conf, jnp.inf)
    kj = key[:, :, None]
    kk = key[:, None, :]
    jj = lax.broadcasted_iota(jnp.int32, (_B, _L, _L), 1)
    kkx = lax.broadcasted_iota(jnp.int32, (_B, _L, _L), 2)
    rank = jnp.sum(
        ((kk < kj) | ((kk == kj) & (kkx < jj))).astype(jnp.float32), axis=2
    )
    neg_sel = cand_mask & (rank < _NUM_NEG_RATIO * num_poses[:, None])

    conf_loss = jnp.sum(_bce(pos_logit, 1.0) * pos_f) + jnp.sum(
        _bce(slot_logit, 0.0) * neg_sel.astype(jnp.float32)
    )

    max_lens = jnp.maximum(l2, l3)
    safe = jnp.where(pos_mask, max_lens, 1.0)
    reg = (
        _sl1(jnp.where(condA, A[:, 48 : 48 + _L], Bh[:, 48 : 48 + _L]) - l0)
        + _sl1(jnp.where(condA, A[:, 64 : 64 + _L], Bh[:, 64 : 64 + _L]) - l1)
        + _sl1(jnp.where(condA, A[:, 80 : 80 + _L], Bh[:, 80 : 80 + _L]) - l2)
        + _sl1(jnp.where(condA, A[:, 96 : 96 + _L], Bh[:, 96 : 96 + _L]) - l3)
    )
    reg_loss = jnp.sum(reg / safe * pos_f)

    ps = jnp.sum(num_poses)
    has_pos = ps >= 1.0
    denom = jnp.maximum(ps, 1.0)
    loss = (conf_loss + _REG_COEFF * reg_loss) / denom
    z = jnp.float32(0.0)
    o0 = jnp.where(has_pos, loss, z)
    o1 = jnp.where(has_pos, conf_loss / denom, z)
    o2 = jnp.where(has_pos, reg_loss / denom, z)
    o3 = jnp.where(has_pos, ps, z)

    li = lax.broadcasted_iota(jnp.int32, (8, 128), 1)
    out_ref[...] = (
        jnp.where(li == 0, o0, 0.0)
        + jnp.where(li == 1, o1, 0.0)
        + jnp.where(li == 2, o2, 0.0)
        + jnp.where(li == 3, o3, 0.0)
    )


def kernel(labels, outputs):
    outs_t = outputs.transpose(2, 0, 1)
    outf = _get_sc_pass()(outs_t, labels)
    lab_t = labels.transpose(2, 0, 1)
    o = pl.pallas_call(
        _epi_body,
        out_shape=jax.ShapeDtypeStruct((8, 128), jnp.float32),
    )(lab_t, outf)
    return o[0, 0], o[0, 1], o[0, 2], o[0, 3]
